# baseline (speedup 1.0000x reference)
import functools

import jax
import jax.numpy as jnp
from jax import lax
from jax.experimental import pallas as pl
from jax.experimental.pallas import tpu as pltpu
from jax.experimental.pallas import tpu_sc as plsc

N_NODES = 10000
D = 256
E = 160000
W = 64
N_CHUNKS = E // W
NS = 16
NC = 2
DH = D // NC
LANES = 16
CPT = (-(-N_CHUNKS // NS) + 7) // 8 * 8
PAD_CHUNKS = CPT * NS
NBUF = 3
ZCH = N_NODES // W
ZTAIL = N_NODES - ZCH * W


def kernel(messages, edge_index):
    dst = edge_index[1].astype(jnp.int32).reshape(N_CHUNKS, W)
    dst = jnp.pad(dst, ((0, PAD_CHUNKS - N_CHUNKS), (0, 0)))
    mesh = plsc.VectorSubcoreMesh(core_axis_name="c", subcore_axis_name="s")

    @functools.partial(
        pl.kernel,
        out_type=jax.ShapeDtypeStruct((N_NODES, D), jnp.float32),
        mesh=mesh,
        scratch_types=[
            pltpu.VMEM((CPT, W), jnp.int32),
            pltpu.VMEM((NBUF, W, DH), jnp.float32),
            pltpu.VMEM_SHARED((N_NODES, DH), jnp.float32),
            pltpu.SemaphoreType.DMA,
            pltpu.SemaphoreType.DMA,
            pltpu.SemaphoreType.DMA,
        ],
    )
    def sc_kernel(msg_hbm, dst_hbm, out_hbm, din_v, rows_v, acc,
                  sem0, sem1, sem2):
        c = lax.axis_index("c")
        s = lax.axis_index("s")
        col = c * DH
        start = s * CPT
        sems = [sem0, sem1, sem2]

        def load(buf, k):
            return pltpu.make_async_copy(
                msg_hbm.at[pl.ds(k * W, W), pl.ds(col, DH)],
                rows_v.at[buf], sems[buf])

        for j in range(NBUF - 1):
            @pl.when(start + j < N_CHUNKS)
            def _():
                load(j + 1, start + j).start()

        pltpu.sync_copy(dst_hbm.at[pl.ds(start, CPT)], din_v)

        def zrow(i, carry):
            r = i // (DH // LANES)
            j = i % (DH // LANES)
            rows_v[0, r, pl.ds(j * LANES, LANES)] = (
                jnp.zeros((LANES,), jnp.float32))
            return carry
        lax.fori_loop(0, W * (DH // LANES), zrow, 0)

        for kk in range((ZCH + NS - 1) // NS):
            k = s + NS * kk
            @pl.when(k < ZCH)
            def _():
                pltpu.sync_copy(rows_v.at[0], acc.at[pl.ds(k * W, W)])
        @pl.when(s == 0)
        def _():
            pltpu.sync_copy(rows_v.at[0, pl.ds(0, ZTAIL)],
                            acc.at[pl.ds(ZCH * W, ZTAIL)])
        plsc.subcore_barrier()

        @pl.when(start + NBUF - 1 < N_CHUNKS)
        def _():
            load(0, start + NBUF - 1).start()

        def outer(o, carry):
            for b in range(NBUF):
                i = o * NBUF + b
                bu = (b + 1) % NBUF
                k = start + i
                @pl.when((i < CPT) & (k < N_CHUNKS))
                def _():
                    load(bu, k).wait()
                    pltpu.sync_copy(rows_v.at[bu], acc.at[din_v.at[i]],
                                    add=True)
                    @pl.when((i + NBUF < CPT) & (k + NBUF < N_CHUNKS))
                    def _():
                        load(bu, k + NBUF).start()
            return carry
        lax.fori_loop(0, (CPT + NBUF - 1) // NBUF, outer, 0)

        plsc.subcore_barrier()

        for kk in range((ZCH + NS - 1) // NS):
            k = s + NS * kk
            @pl.when(k < ZCH)
            def _():
                pltpu.sync_copy(acc.at[pl.ds(k * W, W)],
                                out_hbm.at[pl.ds(k * W, W), pl.ds(col, DH)])
        @pl.when(s == 0)
        def _():
            pltpu.sync_copy(acc.at[pl.ds(ZCH * W, ZTAIL)],
                            out_hbm.at[pl.ds(ZCH * W, ZTAIL), pl.ds(col, DH)])

    return sc_kernel(messages, dst)
